# baseline (speedup 1.0000x reference)
import functools

import jax
import jax.numpy as jnp
from jax import lax
from jax.experimental import pallas as pl
from jax.experimental.pallas import tpu as pltpu
from jax.experimental.pallas import tpu_sc as plsc

NUM_EMBED = 1000000
EMBED_DIM = 64
COL_DIM = 16
BATCH = 16384

_info = plsc.get_sparse_core_info()
NC, NS, L = _info.num_cores, _info.num_subcores, _info.num_lanes
NW = NC * NS
B_PER_W = BATCH // NW
CHUNK = 128
NCHUNK = B_PER_W // CHUNK


def _make_kernel():
    mesh = plsc.VectorSubcoreMesh(core_axis_name="c", subcore_axis_name="s")

    @functools.partial(
        pl.kernel,
        mesh=mesh,
        out_type=jax.ShapeDtypeStruct((BATCH, EMBED_DIM), jnp.float32),
        compiler_params=pltpu.CompilerParams(use_tc_tiling_on_sc=False),
        scratch_types=[
            pltpu.VMEM((NCHUNK, CHUNK), jnp.int32),
            pltpu.VMEM((B_PER_W, EMBED_DIM), jnp.float32),
            pltpu.VMEM((1, COL_DIM), jnp.float32),
            pltpu.SemaphoreType.DMA,
        ],
    )
    def k(x_hbm, table_hbm, shared_hbm, out_hbm, idx_v, rows_v, sh_v, sem):
        wid = lax.axis_index("s") * NC + lax.axis_index("c")
        base = wid * B_PER_W

        for j in range(NCHUNK):
            pltpu.sync_copy(x_hbm.at[pl.ds(base + j * CHUNK, CHUNK)], idx_v.at[j])
        pltpu.sync_copy(shared_hbm, sh_v)

        copies = []
        for j in range(NCHUNK):
            copies.append(
                pltpu.async_copy(
                    table_hbm.at[idx_v.at[j]],
                    rows_v.at[pl.ds(j * CHUNK, CHUNK)],
                    sem,
                )
            )
        for c in copies:
            c.wait()

        sh = sh_v[0, :]

        def body(i, _):
            rows_v[i, 0:COL_DIM] = sh
            return 0

        lax.fori_loop(0, B_PER_W, body, 0)

        pltpu.sync_copy(rows_v, out_hbm.at[pl.ds(base, B_PER_W)])

    return k


_kernel = _make_kernel()


def kernel(X, embed_weight, shared_embed):
    return _kernel(X.astype(jnp.int32), embed_weight, shared_embed)
